# Initial kernel scaffold; baseline (speedup 1.0000x reference)
import jax
import jax.numpy as jnp
from jax.experimental import pallas as pl


def kernel(latents, prototypes):
    raise NotImplementedError("write your pallas kernel here")


# speedup vs baseline: 10.0181x; 10.0181x over previous
import functools

import jax
import jax.numpy as jnp
from jax import lax
from jax.experimental import pallas as pl
from jax.experimental.pallas import tpu as pltpu
from jax.experimental.pallas import tpu_sc as plsc

_K = 8192
_D = 256
_N = 32768
_BN = 256
_NB = _N // _BN
_BETA = 0.25


def _argmin_body(l_ref, pt_ref, idx_ref, msum_ref):
    i = pl.program_id(0)
    l = l_ref[...]
    pt = pt_ref[...]
    lsq = jnp.sum(l * l, axis=1, keepdims=True)
    psq = jnp.sum(pt * pt, axis=0, keepdims=True)
    mm = lax.dot_general(l, pt, (((1,), (0,)), ((), ())),
                         preferred_element_type=jnp.float32)
    d = (lsq + psq) - 2.0 * mm
    minv = jnp.min(d, axis=1, keepdims=True)
    cols = lax.broadcasted_iota(jnp.int32, (_BN, _K), 1)
    idx = jnp.min(jnp.where(d == minv, cols, _K), axis=1, keepdims=True)
    idx_ref[...] = idx

    @pl.when(i == 0)
    def _():
        msum_ref[0, 0] = 0.0

    msum_ref[0, 0] += jnp.sum(minv)


def _tc_argmin(latents, pt):
    return pl.pallas_call(
        _argmin_body,
        grid=(_NB,),
        in_specs=[pl.BlockSpec((_BN, _D), lambda i: (i, 0)),
                  pl.BlockSpec((_D, _K), lambda i: (0, 0))],
        out_specs=[pl.BlockSpec((_BN, 1), lambda i: (i, 0)),
                   pl.BlockSpec(memory_space=pltpu.SMEM)],
        out_shape=[jax.ShapeDtypeStruct((_N, 1), jnp.int32),
                   jax.ShapeDtypeStruct((1, 1), jnp.float32)],
        compiler_params=pltpu.CompilerParams(
            dimension_semantics=("arbitrary",)),
    )(latents, pt)


_NW = 32
_BPW = _N // _NW
_CH = 128


def _sc_gather(table, idx):
    mesh = plsc.VectorSubcoreMesh(core_axis_name="c", subcore_axis_name="s")

    @functools.partial(
        pl.kernel, mesh=mesh,
        out_type=jax.ShapeDtypeStruct((_N, _D), jnp.float32),
        scratch_types=[pltpu.VMEM((_CH,), jnp.int32),
                       pltpu.VMEM((_CH, _D), jnp.float32),
                       pltpu.SemaphoreType.DMA],
    )
    def gather_kernel(table_hbm, idx_hbm, out_hbm, idx_v, rows_v, sem):
        wid = lax.axis_index("s") * 2 + lax.axis_index("c")
        base = wid * _BPW

        @pl.loop(0, _BPW, step=_CH)
        def _(off):
            pltpu.sync_copy(idx_hbm.at[pl.ds(base + off, _CH)], idx_v)
            pltpu.async_copy(table_hbm.at[idx_v], rows_v, sem).wait()
            pltpu.sync_copy(rows_v, out_hbm.at[pl.ds(base + off, _CH)])

    return gather_kernel(table, idx)


def kernel(latents, prototypes):
    pt = prototypes.T
    idx_col, msum = _tc_argmin(latents, pt)
    idx = idx_col.reshape(_N)
    quantized = _sc_gather(prototypes, idx)
    vq_loss = msum[0, 0] * ((1.0 + _BETA) / (_N * _D))
    return quantized, vq_loss


# speedup vs baseline: 12.2436x; 1.2222x over previous
import functools

import jax
import jax.numpy as jnp
from jax import lax
from jax.experimental import pallas as pl
from jax.experimental.pallas import tpu as pltpu
from jax.experimental.pallas import tpu_sc as plsc

_K = 8192
_D = 256
_N = 32768
_BN = 256
_NB = _N // _BN
_BETA = 0.25


def _argmin_body(l_ref, pt_ref, idx_ref, msum_ref, d_ref):
    i = pl.program_id(0)
    l = l_ref[...]
    pt = pt_ref[...]
    lsq = jnp.sum(l * l, axis=1, keepdims=True)
    psq = jnp.sum(pt * pt, axis=0, keepdims=True)
    mm2 = lax.dot_general(2.0 * l, pt, (((1,), (0,)), ((), ())),
                          preferred_element_type=jnp.float32)
    d_ref[...] = (lsq + psq) - mm2
    d = d_ref[...]
    minv = jnp.min(d, axis=1, keepdims=True)
    cols = lax.broadcasted_iota(jnp.int32, (1, _K), 1).astype(jnp.float32)
    idxf = jnp.min(jnp.where(d == minv, cols, 65536.0), axis=1, keepdims=True)
    idx_ref[...] = jnp.minimum(idxf, float(_K - 1)).astype(jnp.int32)

    @pl.when(i == 0)
    def _():
        msum_ref[0, 0] = 0.0

    msum_ref[0, 0] += jnp.sum(minv)


def _tc_argmin(latents, pt):
    return pl.pallas_call(
        _argmin_body,
        grid=(_NB,),
        in_specs=[pl.BlockSpec((_BN, _D), lambda i: (i, 0)),
                  pl.BlockSpec((_D, _K), lambda i: (0, 0))],
        out_specs=[pl.BlockSpec((_BN, 1), lambda i: (i, 0)),
                   pl.BlockSpec(memory_space=pltpu.SMEM)],
        out_shape=[jax.ShapeDtypeStruct((_N, 1), jnp.int32),
                   jax.ShapeDtypeStruct((1, 1), jnp.float32)],
        scratch_shapes=[pltpu.VMEM((_BN, _K), jnp.float32)],
        compiler_params=pltpu.CompilerParams(
            dimension_semantics=("arbitrary",)),
    )(latents, pt)


_NW = 32
_BPW = _N // _NW
_CH = 128


def _sc_gather(table, idx):
    mesh = plsc.VectorSubcoreMesh(core_axis_name="c", subcore_axis_name="s")

    @functools.partial(
        pl.kernel, mesh=mesh,
        out_type=jax.ShapeDtypeStruct((_N, _D), jnp.float32),
        scratch_types=[pltpu.VMEM((_CH,), jnp.int32),
                       pltpu.VMEM((_CH, _D), jnp.float32),
                       pltpu.SemaphoreType.DMA],
    )
    def gather_kernel(table_hbm, idx_hbm, out_hbm, idx_v, rows_v, sem):
        wid = lax.axis_index("s") * 2 + lax.axis_index("c")
        base = wid * _BPW

        @pl.loop(0, _BPW, step=_CH)
        def _(off):
            pltpu.sync_copy(idx_hbm.at[pl.ds(base + off, _CH)], idx_v)
            pltpu.async_copy(table_hbm.at[idx_v], rows_v, sem).wait()
            pltpu.sync_copy(rows_v, out_hbm.at[pl.ds(base + off, _CH)])

    return gather_kernel(table, idx)


def kernel(latents, prototypes):
    pt = prototypes.T
    idx_col, msum = _tc_argmin(latents, pt)
    idx = idx_col.reshape(_N)
    quantized = _sc_gather(prototypes, idx)
    vq_loss = msum[0, 0] * ((1.0 + _BETA) / (_N * _D))
    return quantized, vq_loss
